# Initial kernel scaffold; baseline (speedup 1.0000x reference)
import jax
import jax.numpy as jnp
from jax.experimental import pallas as pl


def kernel(x, edge_index, W_in, b_in, W0, b0, g0, be0, W1, b1, g1, be1, W2, b2, g2, be2):
    raise NotImplementedError("write your pallas kernel here")


# speedup vs baseline: 10.9037x; 10.9037x over previous
import functools

import jax
import jax.numpy as jnp
from jax import lax
from jax.experimental import pallas as pl
from jax.experimental.pallas import tpu as pltpu
from jax.experimental.pallas import tpu_sc as plsc

N = 10000
E = 320000
D = 128
NC, NS = 2, 16
NW = NC * NS
N_PAD = 10240
RPT = N_PAD // NS
EPW = E // NW
CH = 80
NCHUNK = EPW // CH

_mesh = plsc.VectorSubcoreMesh(core_axis_name="c", subcore_axis_name="s")


@functools.partial(
    pl.kernel,
    out_type=jax.ShapeDtypeStruct((NC * N_PAD,), jnp.float32),
    mesh=_mesh,
    scratch_types=[
        pltpu.VMEM((CH,), jnp.int32),
        pltpu.VMEM((CH,), jnp.float32),
        pltpu.VMEM_SHARED((N_PAD,), jnp.float32),
    ],
)
def _deg_kernel(dst_hbm, zero1_hbm, out_hbm, idx_v, ones_v, acc_sh):
    cid = lax.axis_index("c")
    sid = lax.axis_index("s")
    wid = cid * NS + sid
    pltpu.sync_copy(zero1_hbm.at[pl.ds(sid * RPT, RPT)],
                    acc_sh.at[pl.ds(sid * RPT, RPT)])
    for i in range(CH // 16):
        ones_v[pl.ds(i * 16, 16)] = jnp.ones((16,), jnp.float32)
    plsc.subcore_barrier()

    def body(k, carry):
        off = pl.multiple_of(wid * EPW + k * CH, 8)
        pltpu.sync_copy(dst_hbm.at[pl.ds(off, CH)], idx_v)
        pltpu.sync_copy(ones_v, acc_sh.at[idx_v], add=True)
        return carry

    lax.fori_loop(0, NCHUNK, body, 0)
    plsc.subcore_barrier()
    pltpu.sync_copy(acc_sh.at[pl.ds(sid * RPT, RPT)],
                    out_hbm.at[pl.ds(cid * N_PAD + sid * RPT, RPT)])


@functools.partial(
    pl.kernel,
    out_type=jax.ShapeDtypeStruct((NC * N_PAD, D), jnp.float32),
    mesh=_mesh,
    scratch_types=[
        pltpu.VMEM((CH,), jnp.int32),
        pltpu.VMEM((CH,), jnp.int32),
        pltpu.VMEM((CH, D), jnp.float32),
        pltpu.VMEM_SHARED((N_PAD, D), jnp.float32),
        pltpu.SemaphoreType.DMA,
    ],
)
def _edge_sum_kernel(src_hbm, dst_hbm, table_hbm, zeros_hbm, out_hbm,
                     sidx_v, didx_v, rows_v, acc_sh, sem):
    cid = lax.axis_index("c")
    sid = lax.axis_index("s")
    wid = cid * NS + sid
    pltpu.sync_copy(zeros_hbm.at[pl.ds(sid * RPT, RPT)],
                    acc_sh.at[pl.ds(sid * RPT, RPT)])
    plsc.subcore_barrier()

    def body(k, carry):
        off = pl.multiple_of(wid * EPW + k * CH, 8)
        pltpu.sync_copy(src_hbm.at[pl.ds(off, CH)], sidx_v)
        pltpu.sync_copy(dst_hbm.at[pl.ds(off, CH)], didx_v)
        pltpu.async_copy(table_hbm.at[sidx_v], rows_v, sem).wait()
        pltpu.sync_copy(rows_v, acc_sh.at[didx_v], add=True)
        return carry

    lax.fori_loop(0, NCHUNK, body, 0)
    plsc.subcore_barrier()
    pltpu.sync_copy(acc_sh.at[pl.ds(sid * RPT, RPT)],
                    out_hbm.at[pl.ds(cid * N_PAD + sid * RPT, RPT)])


BLK = 640
GRID = N_PAD // BLK

_row_spec = pl.BlockSpec((BLK, D), lambda i: (i, 0))
_w_spec = pl.BlockSpec((D, D), lambda i: (0, 0))
_vec_spec = pl.BlockSpec((1, D), lambda i: (0, 0))
_dinv_spec = pl.BlockSpec((BLK, 1), lambda i: (i, 0))
_s0_spec = pl.BlockSpec((BLK, D), lambda i: (i, 0))
_s1_spec = pl.BlockSpec((BLK, D), lambda i: (GRID + i, 0))


def _tc0_body(degp_ref, x_ref, win_ref, bin_ref, w0_ref,
              h0_ref, hT0_ref, dinv_ref):
    deg = degp_ref[:, 0:1] + degp_ref[:, 1:2] + 1.0
    dinv = lax.rsqrt(deg)
    h0 = jnp.dot(x_ref[...], win_ref[...],
                 preferred_element_type=jnp.float32) + bin_ref[...]
    h0_ref[...] = h0
    hT0_ref[...] = dinv * jnp.dot(h0, w0_ref[...],
                                  preferred_element_type=jnp.float32)
    dinv_ref[...] = dinv


_tc0 = pl.pallas_call(
    _tc0_body,
    grid=(GRID,),
    in_specs=[pl.BlockSpec((BLK, NC), lambda i: (i, 0)),
              _row_spec, _w_spec, _vec_spec, _w_spec],
    out_specs=[_row_spec, _row_spec, _dinv_spec],
    out_shape=[jax.ShapeDtypeStruct((N_PAD, D), jnp.float32),
               jax.ShapeDtypeStruct((N_PAD, D), jnp.float32),
               jax.ShapeDtypeStruct((N_PAD, 1), jnp.float32)],
)


def _post_agg(s0, s1, hT, h, dinv, b, g, be):
    agg = dinv * (s0 + s1 + hT) + b
    mu = jnp.mean(agg, axis=-1, keepdims=True)
    xc = agg - mu
    var = jnp.mean(xc * xc, axis=-1, keepdims=True)
    ln = xc * lax.rsqrt(var + 1e-5) * g + be
    return jnp.maximum(ln, 0.0) + h


def _mid_body(s_ref, s1_ref, hT_ref, h_ref, dinv_ref, b_ref, g_ref, be_ref,
              wn_ref, h1_ref, hT1_ref):
    dinv = dinv_ref[...]
    hn = _post_agg(s_ref[...], s1_ref[...], hT_ref[...], h_ref[...],
                   dinv, b_ref[...], g_ref[...], be_ref[...])
    h1_ref[...] = hn
    hT1_ref[...] = dinv * jnp.dot(hn, wn_ref[...],
                                  preferred_element_type=jnp.float32)


_tc_mid = pl.pallas_call(
    _mid_body,
    grid=(GRID,),
    in_specs=[_s0_spec, _s1_spec, _row_spec, _row_spec, _dinv_spec,
              _vec_spec, _vec_spec, _vec_spec, _w_spec],
    out_specs=[_row_spec, _row_spec],
    out_shape=[jax.ShapeDtypeStruct((N_PAD, D), jnp.float32),
               jax.ShapeDtypeStruct((N_PAD, D), jnp.float32)],
)


def _fin_body(s_ref, s1_ref, hT_ref, h_ref, dinv_ref, b_ref, g_ref, be_ref,
              h1_ref):
    h1_ref[...] = _post_agg(s_ref[...], s1_ref[...], hT_ref[...], h_ref[...],
                            dinv_ref[...], b_ref[...], g_ref[...], be_ref[...])


_tc_fin = pl.pallas_call(
    _fin_body,
    grid=(GRID,),
    in_specs=[_s0_spec, _s1_spec, _row_spec, _row_spec, _dinv_spec,
              _vec_spec, _vec_spec, _vec_spec],
    out_specs=_row_spec,
    out_shape=jax.ShapeDtypeStruct((N_PAD, D), jnp.float32),
)


def kernel(x, edge_index, W_in, b_in, W0, b0, g0, be0,
           W1, b1, g1, be1, W2, b2, g2, be2):
    src = edge_index[0]
    dst = edge_index[1]
    xp = jnp.pad(x, ((0, N_PAD - N), (0, 0)))
    zeros2 = jnp.zeros((N_PAD, D), jnp.float32)
    zeros1 = jnp.zeros((N_PAD,), jnp.float32)

    degp = _deg_kernel(dst, zeros1)
    degp2 = jnp.transpose(degp.reshape(NC, N_PAD))

    h, hT, dinv = _tc0(degp2, xp, W_in, b_in.reshape(1, D), W0)

    for (b, g, be, Wn) in ((b0, g0, be0, W1), (b1, g1, be1, W2),
                           (b2, g2, be2, None)):
        s = _edge_sum_kernel(src, dst, hT, zeros2)
        if Wn is None:
            h = _tc_fin(s, s, hT, h, dinv, b.reshape(1, D), g.reshape(1, D),
                        be.reshape(1, D))
        else:
            h, hT = _tc_mid(s, s, hT, h, dinv, b.reshape(1, D),
                            g.reshape(1, D), be.reshape(1, D), Wn)
    return h[:N]


# speedup vs baseline: 27.9728x; 2.5654x over previous
import functools

import jax
import jax.numpy as jnp
from jax import lax
from jax.experimental import pallas as pl
from jax.experimental.pallas import tpu as pltpu
from jax.experimental.pallas import tpu_sc as plsc

N = 10000
E = 320000
D = 128
NC, NS = 2, 16
NW = NC * NS
N_PAD = 10240
RPT = N_PAD // NS
EPW = E // NW
CH = 80
NCHUNK = EPW // CH

NBUF = 4
LOOKAHEAD = 2
IBUF = 8
ILOOK = 5

_mesh = plsc.VectorSubcoreMesh(core_axis_name="c", subcore_axis_name="s")


@functools.partial(
    pl.kernel,
    out_type=jax.ShapeDtypeStruct((NC * N_PAD,), jnp.float32),
    mesh=_mesh,
    scratch_types=[
        pltpu.VMEM((NCHUNK, CH), jnp.int32),
        pltpu.VMEM((CH,), jnp.float32),
        pltpu.VMEM_SHARED((N_PAD,), jnp.float32),
        pltpu.SemaphoreType.DMA,
    ],
)
def _deg_kernel(dst_hbm, zero1_hbm, out_hbm, didx_v, ones_v, acc_sh, sem):
    cid = lax.axis_index("c")
    sid = lax.axis_index("s")
    wid = cid * NS + sid
    pltpu.sync_copy(dst_hbm.at[wid], didx_v)
    pltpu.sync_copy(zero1_hbm.at[pl.ds(sid * RPT, RPT)],
                    acc_sh.at[pl.ds(sid * RPT, RPT)])
    for i in range(CH // 16):
        ones_v[pl.ds(i * 16, 16)] = jnp.ones((16,), jnp.float32)
    plsc.subcore_barrier()

    for k in range(5):
        pltpu.async_copy(ones_v, acc_sh.at[didx_v.at[k]], sem, add=True)

    def body(k, carry):
        pltpu.async_copy(ones_v, acc_sh.at[didx_v.at[k + 5]], sem, add=True)
        pltpu.make_async_copy(ones_v, acc_sh.at[didx_v.at[k]], sem).wait()
        return carry

    lax.fori_loop(0, NCHUNK - 5, body, 0)
    for k in range(NCHUNK - 5, NCHUNK):
        pltpu.make_async_copy(ones_v, acc_sh.at[didx_v.at[k]], sem).wait()
    plsc.subcore_barrier()
    pltpu.sync_copy(acc_sh.at[pl.ds(sid * RPT, RPT)],
                    out_hbm.at[pl.ds(cid * N_PAD + sid * RPT, RPT)])


@functools.partial(
    pl.kernel,
    out_type=jax.ShapeDtypeStruct((NC * N_PAD, D), jnp.float32),
    mesh=_mesh,
    scratch_types=[
        pltpu.VMEM((IBUF, CH), jnp.int32),
        pltpu.VMEM((IBUF, CH), jnp.int32),
        pltpu.VMEM((NBUF, CH, D), jnp.float32),
        pltpu.VMEM_SHARED((N_PAD, D), jnp.float32),
        pltpu.SemaphoreType.DMA,
        pltpu.SemaphoreType.DMA,
        pltpu.SemaphoreType.DMA,
    ],
)
def _edge_sum_kernel(src_hbm, dst_hbm, table_hbm, zeros_hbm, out_hbm,
                     sidx_v, didx_v, rows_v, acc_sh, gsem, ssem, isem):
    cid = lax.axis_index("c")
    sid = lax.axis_index("s")
    wid = cid * NS + sid

    def idx_issue(k):
        i = lax.rem(k, IBUF) if not isinstance(k, int) else k % IBUF
        pltpu.async_copy(src_hbm.at[wid].at[k], sidx_v.at[i], isem)
        pltpu.async_copy(dst_hbm.at[wid].at[k], didx_v.at[i], isem)

    def idx_wait(k):
        i = lax.rem(k, IBUF) if not isinstance(k, int) else k % IBUF
        pltpu.make_async_copy(src_hbm.at[wid].at[k], sidx_v.at[i],
                              isem).wait()
        pltpu.make_async_copy(dst_hbm.at[wid].at[k], didx_v.at[i],
                              isem).wait()

    def gather(k, slot):
        i = lax.rem(k, IBUF) if not isinstance(k, int) else k % IBUF
        return pltpu.async_copy(table_hbm.at[sidx_v.at[i]], rows_v.at[slot],
                                gsem)

    def gather_wait(k, slot):
        i = lax.rem(k, IBUF) if not isinstance(k, int) else k % IBUF
        pltpu.make_async_copy(table_hbm.at[sidx_v.at[i]], rows_v.at[slot],
                              gsem).wait()

    def scat(k, slot):
        i = lax.rem(k, IBUF) if not isinstance(k, int) else k % IBUF
        return pltpu.async_copy(rows_v.at[slot], acc_sh.at[didx_v.at[i]],
                                ssem, add=True)

    def scat_wait(k, slot):
        i = lax.rem(k, IBUF) if not isinstance(k, int) else k % IBUF
        pltpu.make_async_copy(rows_v.at[slot], acc_sh.at[didx_v.at[i]],
                              ssem).wait()

    for k in range(ILOOK):
        idx_issue(k)
    for k in range(LOOKAHEAD):
        idx_wait(k)
        gather(k, k)
    pltpu.sync_copy(zeros_hbm.at[pl.ds(sid * RPT, RPT)],
                    acc_sh.at[pl.ds(sid * RPT, RPT)])
    plsc.subcore_barrier()

    def body(k, carry):
        slot = lax.rem(k, NBUF)
        gather_wait(k, slot)
        scat(k, slot)

        @pl.when(k >= LOOKAHEAD)
        def _():
            kd = k - LOOKAHEAD
            scat_wait(kd, lax.rem(kd, NBUF))

        @pl.when(k < NCHUNK - ILOOK)
        def _():
            idx_issue(k + ILOOK)

        @pl.when(k < NCHUNK - LOOKAHEAD)
        def _():
            kn = k + LOOKAHEAD
            idx_wait(kn)
            gather(kn, lax.rem(kn, NBUF))
        return carry

    lax.fori_loop(0, NCHUNK, body, 0)
    for k in range(NCHUNK - LOOKAHEAD, NCHUNK):
        scat_wait(k, k % NBUF)
    plsc.subcore_barrier()
    pltpu.sync_copy(acc_sh.at[pl.ds(sid * RPT, RPT)],
                    out_hbm.at[pl.ds(cid * N_PAD + sid * RPT, RPT)])


BLK = 640
GRID = N_PAD // BLK

_row_spec = pl.BlockSpec((BLK, D), lambda i: (i, 0))
_w_spec = pl.BlockSpec((D, D), lambda i: (0, 0))
_vec_spec = pl.BlockSpec((1, D), lambda i: (0, 0))
_dinv_spec = pl.BlockSpec((BLK, 1), lambda i: (i, 0))
_s0_spec = pl.BlockSpec((BLK, D), lambda i: (i, 0))
_s1_spec = pl.BlockSpec((BLK, D), lambda i: (GRID + i, 0))


def _tc0_body(degp_ref, x_ref, win_ref, bin_ref, w0_ref,
              h0_ref, hT0_ref, dinv_ref):
    deg = degp_ref[:, 0:1] + degp_ref[:, 1:2] + 1.0
    dinv = lax.rsqrt(deg)
    h0 = jnp.dot(x_ref[...], win_ref[...],
                 preferred_element_type=jnp.float32) + bin_ref[...]
    h0_ref[...] = h0
    hT0_ref[...] = dinv * jnp.dot(h0, w0_ref[...],
                                  preferred_element_type=jnp.float32)
    dinv_ref[...] = dinv


_tc0 = pl.pallas_call(
    _tc0_body,
    grid=(GRID,),
    in_specs=[pl.BlockSpec((BLK, NC), lambda i: (i, 0)),
              _row_spec, _w_spec, _vec_spec, _w_spec],
    out_specs=[_row_spec, _row_spec, _dinv_spec],
    out_shape=[jax.ShapeDtypeStruct((N_PAD, D), jnp.float32),
               jax.ShapeDtypeStruct((N_PAD, D), jnp.float32),
               jax.ShapeDtypeStruct((N_PAD, 1), jnp.float32)],
)


def _post_agg(s0, s1, hT, h, dinv, b, g, be):
    agg = dinv * (s0 + s1 + hT) + b
    mu = jnp.mean(agg, axis=-1, keepdims=True)
    xc = agg - mu
    var = jnp.mean(xc * xc, axis=-1, keepdims=True)
    ln = xc * lax.rsqrt(var + 1e-5) * g + be
    return jnp.maximum(ln, 0.0) + h


def _mid_body(s_ref, s1_ref, hT_ref, h_ref, dinv_ref, b_ref, g_ref, be_ref,
              wn_ref, h1_ref, hT1_ref):
    dinv = dinv_ref[...]
    hn = _post_agg(s_ref[...], s1_ref[...], hT_ref[...], h_ref[...],
                   dinv, b_ref[...], g_ref[...], be_ref[...])
    h1_ref[...] = hn
    hT1_ref[...] = dinv * jnp.dot(hn, wn_ref[...],
                                  preferred_element_type=jnp.float32)


_tc_mid = pl.pallas_call(
    _mid_body,
    grid=(GRID,),
    in_specs=[_s0_spec, _s1_spec, _row_spec, _row_spec, _dinv_spec,
              _vec_spec, _vec_spec, _vec_spec, _w_spec],
    out_specs=[_row_spec, _row_spec],
    out_shape=[jax.ShapeDtypeStruct((N_PAD, D), jnp.float32),
               jax.ShapeDtypeStruct((N_PAD, D), jnp.float32)],
)


def _fin_body(s_ref, s1_ref, hT_ref, h_ref, dinv_ref, b_ref, g_ref, be_ref,
              h1_ref):
    h1_ref[...] = _post_agg(s_ref[...], s1_ref[...], hT_ref[...], h_ref[...],
                            dinv_ref[...], b_ref[...], g_ref[...], be_ref[...])


_tc_fin = pl.pallas_call(
    _fin_body,
    grid=(GRID,),
    in_specs=[_s0_spec, _s1_spec, _row_spec, _row_spec, _dinv_spec,
              _vec_spec, _vec_spec, _vec_spec],
    out_specs=_row_spec,
    out_shape=jax.ShapeDtypeStruct((N_PAD, D), jnp.float32),
)


def kernel(x, edge_index, W_in, b_in, W0, b0, g0, be0,
           W1, b1, g1, be1, W2, b2, g2, be2):
    src = edge_index[0].reshape(NW, NCHUNK, CH)
    dst = edge_index[1].reshape(NW, NCHUNK, CH)
    xp = jnp.pad(x, ((0, N_PAD - N), (0, 0)))
    zeros2 = jnp.zeros((N_PAD, D), jnp.float32)
    zeros1 = jnp.zeros((N_PAD,), jnp.float32)

    degp = _deg_kernel(dst, zeros1)
    degp2 = jnp.transpose(degp.reshape(NC, N_PAD))

    h, hT, dinv = _tc0(degp2, xp, W_in, b_in.reshape(1, D), W0)

    for (b, g, be, Wn) in ((b0, g0, be0, W1), (b1, g1, be1, W2),
                           (b2, g2, be2, None)):
        s = _edge_sum_kernel(src, dst, hT, zeros2)
        if Wn is None:
            h = _tc_fin(s, s, hT, h, dinv, b.reshape(1, D), g.reshape(1, D),
                        be.reshape(1, D))
        else:
            h, hT = _tc_mid(s, s, hT, h, dinv, b.reshape(1, D),
                            g.reshape(1, D), be.reshape(1, D), Wn)
    return h[:N]


# speedup vs baseline: 28.3519x; 1.0136x over previous
import functools

import jax
import jax.numpy as jnp
from jax import lax
from jax.experimental import pallas as pl
from jax.experimental.pallas import tpu as pltpu
from jax.experimental.pallas import tpu_sc as plsc

N = 10000
E = 320000
D = 128
NC, NS = 2, 16
NW = NC * NS
N_PAD = 10240
RPT = N_PAD // NS

EPW = E // NW
CH = 80
NCHUNK = EPW // CH

NBUF = 4
LOOKAHEAD = 2
IBUF = 8
ILOOK = 5

_mesh = plsc.VectorSubcoreMesh(core_axis_name="c", subcore_axis_name="s")


@functools.partial(
    pl.kernel,
    out_type=jax.ShapeDtypeStruct((NC * N_PAD,), jnp.float32),
    mesh=_mesh,
    scratch_types=[
        pltpu.VMEM((NCHUNK, CH), jnp.int32),
        pltpu.VMEM((CH,), jnp.float32),
        pltpu.VMEM_SHARED((N_PAD,), jnp.float32),
        pltpu.SemaphoreType.DMA,
    ],
)
def _deg_kernel(dst_hbm, zero1_hbm, out_hbm, didx_v, ones_v, acc_sh, sem):
    cid = lax.axis_index("c")
    sid = lax.axis_index("s")
    wid = cid * NS + sid
    pltpu.sync_copy(dst_hbm.at[wid], didx_v)
    pltpu.sync_copy(zero1_hbm.at[pl.ds(sid * RPT, RPT)],
                    acc_sh.at[pl.ds(sid * RPT, RPT)])
    for i in range(CH // 16):
        ones_v[pl.ds(i * 16, 16)] = jnp.ones((16,), jnp.float32)
    plsc.subcore_barrier()

    for k in range(5):
        pltpu.async_copy(ones_v, acc_sh.at[didx_v.at[k]], sem, add=True)

    def body(k, carry):
        pltpu.async_copy(ones_v, acc_sh.at[didx_v.at[k + 5]], sem, add=True)
        pltpu.make_async_copy(ones_v, acc_sh.at[didx_v.at[k]], sem).wait()
        return carry

    lax.fori_loop(0, NCHUNK - 5, body, 0)
    for k in range(NCHUNK - 5, NCHUNK):
        pltpu.make_async_copy(ones_v, acc_sh.at[didx_v.at[k]], sem).wait()
    plsc.subcore_barrier()
    pltpu.sync_copy(acc_sh.at[pl.ds(sid * RPT, RPT)],
                    out_hbm.at[pl.ds(cid * N_PAD + sid * RPT, RPT)])


@functools.partial(
    pl.kernel,
    out_type=jax.ShapeDtypeStruct((NC * N_PAD, D), jnp.float32),
    mesh=_mesh,
    scratch_types=[
        pltpu.VMEM((IBUF, CH), jnp.int32),
        pltpu.VMEM((IBUF, CH), jnp.int32),
        pltpu.VMEM((NBUF, CH, D), jnp.float32),
        pltpu.VMEM_SHARED((N_PAD, D), jnp.float32),
        pltpu.SemaphoreType.DMA,
        pltpu.SemaphoreType.DMA,
        pltpu.SemaphoreType.DMA,
    ],
)
def _edge_sum_kernel(src_hbm, dst_hbm, table_hbm, zeros_hbm, out_hbm,
                     sidx_v, didx_v, rows_v, acc_sh, gsem, ssem, isem):
    cid = lax.axis_index("c")
    sid = lax.axis_index("s")
    wid = cid * NS + sid

    def idx_issue(k):
        i = lax.rem(k, IBUF) if not isinstance(k, int) else k % IBUF
        pltpu.async_copy(src_hbm.at[wid].at[k], sidx_v.at[i], isem)
        pltpu.async_copy(dst_hbm.at[wid].at[k], didx_v.at[i], isem)

    def idx_wait(k):
        i = lax.rem(k, IBUF) if not isinstance(k, int) else k % IBUF
        pltpu.make_async_copy(src_hbm.at[wid].at[k], sidx_v.at[i],
                              isem).wait()
        pltpu.make_async_copy(dst_hbm.at[wid].at[k], didx_v.at[i],
                              isem).wait()

    def gather(k, slot):
        i = lax.rem(k, IBUF) if not isinstance(k, int) else k % IBUF
        return pltpu.async_copy(table_hbm.at[sidx_v.at[i]], rows_v.at[slot],
                                gsem)

    def gather_wait(k, slot):
        i = lax.rem(k, IBUF) if not isinstance(k, int) else k % IBUF
        pltpu.make_async_copy(table_hbm.at[sidx_v.at[i]], rows_v.at[slot],
                              gsem).wait()

    def scat(k, slot):
        i = lax.rem(k, IBUF) if not isinstance(k, int) else k % IBUF
        return pltpu.async_copy(rows_v.at[slot], acc_sh.at[didx_v.at[i]],
                                ssem, add=True)

    def scat_wait(k, slot):
        i = lax.rem(k, IBUF) if not isinstance(k, int) else k % IBUF
        pltpu.make_async_copy(rows_v.at[slot], acc_sh.at[didx_v.at[i]],
                              ssem).wait()

    for k in range(ILOOK):
        idx_issue(k)
    for k in range(LOOKAHEAD):
        idx_wait(k)
        gather(k, k)
    pltpu.sync_copy(zeros_hbm.at[pl.ds(sid * RPT, RPT)],
                    acc_sh.at[pl.ds(sid * RPT, RPT)])
    plsc.subcore_barrier()

    def body(k, carry):
        slot = lax.rem(k, NBUF)
        gather_wait(k, slot)
        scat(k, slot)

        @pl.when(k >= LOOKAHEAD)
        def _():
            kd = k - LOOKAHEAD
            scat_wait(kd, lax.rem(kd, NBUF))

        @pl.when(k < NCHUNK - ILOOK)
        def _():
            idx_issue(k + ILOOK)

        @pl.when(k < NCHUNK - LOOKAHEAD)
        def _():
            kn = k + LOOKAHEAD
            idx_wait(kn)
            gather(kn, lax.rem(kn, NBUF))
        return carry

    lax.fori_loop(0, NCHUNK, body, 0)
    for k in range(NCHUNK - LOOKAHEAD, NCHUNK):
        scat_wait(k, k % NBUF)
    plsc.subcore_barrier()
    pltpu.sync_copy(acc_sh.at[pl.ds(sid * RPT, RPT)],
                    out_hbm.at[pl.ds(cid * N_PAD + sid * RPT, RPT)])


BLK = 640
GRID = 16
GRID_S = N_PAD // BLK

_row_spec = pl.BlockSpec((BLK, D), lambda i: (i, 0))
_w_spec = pl.BlockSpec((D, D), lambda i: (0, 0))
_vec_spec = pl.BlockSpec((1, D), lambda i: (0, 0))
_dinv_spec = pl.BlockSpec((BLK, 1), lambda i: (i, 0))
_s0_spec = pl.BlockSpec((BLK, D), lambda i: (i, 0))
_s1_spec = pl.BlockSpec((BLK, D), lambda i: (GRID_S + i, 0))


def _tc0_body(degp_ref, x_ref, win_ref, bin_ref, w0_ref,
              h0_ref, hT0_ref, dinv_ref):
    deg = degp_ref[:, 0:1] + degp_ref[:, 1:2] + 1.0
    dinv = lax.rsqrt(deg)
    h0 = jnp.dot(x_ref[...], win_ref[...],
                 preferred_element_type=jnp.float32) + bin_ref[...]
    h0_ref[...] = h0
    hT0_ref[...] = dinv * jnp.dot(h0, w0_ref[...],
                                  preferred_element_type=jnp.float32)
    dinv_ref[...] = dinv


_tc0 = pl.pallas_call(
    _tc0_body,
    grid=(GRID,),
    in_specs=[pl.BlockSpec((BLK, NC), lambda i: (i, 0)),
              _row_spec, _w_spec, _vec_spec, _w_spec],
    out_specs=[_row_spec, _row_spec, _dinv_spec],
    out_shape=[jax.ShapeDtypeStruct((N, D), jnp.float32),
               jax.ShapeDtypeStruct((N, D), jnp.float32),
               jax.ShapeDtypeStruct((N, 1), jnp.float32)],
)


def _post_agg(s0, s1, hT, h, dinv, b, g, be):
    agg = dinv * (s0 + s1 + hT) + b
    mu = jnp.mean(agg, axis=-1, keepdims=True)
    xc = agg - mu
    var = jnp.mean(xc * xc, axis=-1, keepdims=True)
    ln = xc * lax.rsqrt(var + 1e-5) * g + be
    return jnp.maximum(ln, 0.0) + h


def _mid_body(s_ref, s1_ref, hT_ref, h_ref, dinv_ref, b_ref, g_ref, be_ref,
              wn_ref, h1_ref, hT1_ref):
    dinv = dinv_ref[...]
    hn = _post_agg(s_ref[...], s1_ref[...], hT_ref[...], h_ref[...],
                   dinv, b_ref[...], g_ref[...], be_ref[...])
    h1_ref[...] = hn
    hT1_ref[...] = dinv * jnp.dot(hn, wn_ref[...],
                                  preferred_element_type=jnp.float32)


_tc_mid = pl.pallas_call(
    _mid_body,
    grid=(GRID,),
    in_specs=[_s0_spec, _s1_spec, _row_spec, _row_spec, _dinv_spec,
              _vec_spec, _vec_spec, _vec_spec, _w_spec],
    out_specs=[_row_spec, _row_spec],
    out_shape=[jax.ShapeDtypeStruct((N, D), jnp.float32),
               jax.ShapeDtypeStruct((N, D), jnp.float32)],
)


def _fin_body(s_ref, s1_ref, hT_ref, h_ref, dinv_ref, b_ref, g_ref, be_ref,
              h1_ref):
    h1_ref[...] = _post_agg(s_ref[...], s1_ref[...], hT_ref[...], h_ref[...],
                            dinv_ref[...], b_ref[...], g_ref[...], be_ref[...])


_tc_fin = pl.pallas_call(
    _fin_body,
    grid=(GRID,),
    in_specs=[_s0_spec, _s1_spec, _row_spec, _row_spec, _dinv_spec,
              _vec_spec, _vec_spec, _vec_spec],
    out_specs=_row_spec,
    out_shape=jax.ShapeDtypeStruct((N, D), jnp.float32),
)


def kernel(x, edge_index, W_in, b_in, W0, b0, g0, be0,
           W1, b1, g1, be1, W2, b2, g2, be2):
    src = edge_index[0].reshape(NW, NCHUNK, CH)
    dst = edge_index[1].reshape(NW, NCHUNK, CH)
    zeros2 = jnp.zeros((N_PAD, D), jnp.float32)
    zeros1 = jnp.zeros((N_PAD,), jnp.float32)

    degp = _deg_kernel(dst, zeros1)
    degp2 = jnp.transpose(degp.reshape(NC, N_PAD))

    h, hT, dinv = _tc0(degp2, x, W_in, b_in.reshape(1, D), W0)

    for (b, g, be, Wn) in ((b0, g0, be0, W1), (b1, g1, be1, W2),
                           (b2, g2, be2, None)):
        s = _edge_sum_kernel(src, dst, hT, zeros2)
        if Wn is None:
            h = _tc_fin(s, s, hT, h, dinv, b.reshape(1, D), g.reshape(1, D),
                        be.reshape(1, D))
        else:
            h, hT = _tc_mid(s, s, hT, h, dinv, b.reshape(1, D),
                            g.reshape(1, D), be.reshape(1, D), Wn)
    return h


# speedup vs baseline: 28.9027x; 1.0194x over previous
import functools

import jax
import jax.numpy as jnp
from jax import lax
from jax.experimental import pallas as pl
from jax.experimental.pallas import tpu as pltpu
from jax.experimental.pallas import tpu_sc as plsc

N = 10000
E = 320000
D = 128
NC, NS = 2, 16
NW = NC * NS
N_PAD = 10240
RPT = N_PAD // NS

EPW = E // NW
CH = 80
NCHUNK = EPW // CH
CH2 = 40
NCHUNK2 = EPW // CH2

NBUF = 8
LOOKAHEAD = 4
IBUF = 16
ILOOK = 10

_mesh = plsc.VectorSubcoreMesh(core_axis_name="c", subcore_axis_name="s")


@functools.partial(
    pl.kernel,
    out_type=jax.ShapeDtypeStruct((NC * N_PAD,), jnp.float32),
    mesh=_mesh,
    scratch_types=[
        pltpu.VMEM((NCHUNK, CH), jnp.int32),
        pltpu.VMEM((CH,), jnp.float32),
        pltpu.VMEM_SHARED((N_PAD,), jnp.float32),
        pltpu.SemaphoreType.DMA,
    ],
)
def _deg_kernel(dst_hbm, zero1_hbm, out_hbm, didx_v, ones_v, acc_sh, sem):
    cid = lax.axis_index("c")
    sid = lax.axis_index("s")
    wid = cid * NS + sid
    pltpu.sync_copy(dst_hbm.at[wid], didx_v)
    pltpu.sync_copy(zero1_hbm.at[pl.ds(sid * RPT, RPT)],
                    acc_sh.at[pl.ds(sid * RPT, RPT)])
    for i in range(CH // 16):
        ones_v[pl.ds(i * 16, 16)] = jnp.ones((16,), jnp.float32)
    plsc.subcore_barrier()

    for k in range(5):
        pltpu.async_copy(ones_v, acc_sh.at[didx_v.at[k]], sem, add=True)

    def body(k, carry):
        pltpu.async_copy(ones_v, acc_sh.at[didx_v.at[k + 5]], sem, add=True)
        pltpu.make_async_copy(ones_v, acc_sh.at[didx_v.at[k]], sem).wait()
        return carry

    lax.fori_loop(0, NCHUNK - 5, body, 0)
    for k in range(NCHUNK - 5, NCHUNK):
        pltpu.make_async_copy(ones_v, acc_sh.at[didx_v.at[k]], sem).wait()
    plsc.subcore_barrier()
    pltpu.sync_copy(acc_sh.at[pl.ds(sid * RPT, RPT)],
                    out_hbm.at[pl.ds(cid * N_PAD + sid * RPT, RPT)])


@functools.partial(
    pl.kernel,
    out_type=jax.ShapeDtypeStruct((NC * N_PAD, D), jnp.float32),
    mesh=_mesh,
    scratch_types=[
        pltpu.VMEM((IBUF, CH2), jnp.int32),
        pltpu.VMEM((IBUF, CH2), jnp.int32),
        pltpu.VMEM((NBUF, CH2, D), jnp.float32),
        pltpu.VMEM_SHARED((N_PAD, D), jnp.float32),
        pltpu.SemaphoreType.DMA,
        pltpu.SemaphoreType.DMA,
        pltpu.SemaphoreType.DMA,
    ],
)
def _edge_sum_kernel(src_hbm, dst_hbm, table_hbm, zeros_hbm, out_hbm,
                     sidx_v, didx_v, rows_v, acc_sh, gsem, ssem, isem):
    cid = lax.axis_index("c")
    sid = lax.axis_index("s")
    wid = cid * NS + sid

    def idx_issue(k):
        i = lax.rem(k, IBUF) if not isinstance(k, int) else k % IBUF
        pltpu.async_copy(src_hbm.at[wid].at[k], sidx_v.at[i], isem)
        pltpu.async_copy(dst_hbm.at[wid].at[k], didx_v.at[i], isem)

    def idx_wait(k):
        i = lax.rem(k, IBUF) if not isinstance(k, int) else k % IBUF
        pltpu.make_async_copy(src_hbm.at[wid].at[k], sidx_v.at[i],
                              isem).wait()
        pltpu.make_async_copy(dst_hbm.at[wid].at[k], didx_v.at[i],
                              isem).wait()

    def gather(k, slot):
        i = lax.rem(k, IBUF) if not isinstance(k, int) else k % IBUF
        return pltpu.async_copy(table_hbm.at[sidx_v.at[i]], rows_v.at[slot],
                                gsem)

    def gather_wait(k, slot):
        i = lax.rem(k, IBUF) if not isinstance(k, int) else k % IBUF
        pltpu.make_async_copy(table_hbm.at[sidx_v.at[i]], rows_v.at[slot],
                              gsem).wait()

    def scat(k, slot):
        i = lax.rem(k, IBUF) if not isinstance(k, int) else k % IBUF
        return pltpu.async_copy(rows_v.at[slot], acc_sh.at[didx_v.at[i]],
                                ssem, add=True)

    def scat_wait(k, slot):
        i = lax.rem(k, IBUF) if not isinstance(k, int) else k % IBUF
        pltpu.make_async_copy(rows_v.at[slot], acc_sh.at[didx_v.at[i]],
                              ssem).wait()

    for k in range(ILOOK):
        idx_issue(k)
    for k in range(LOOKAHEAD):
        idx_wait(k)
        gather(k, k)
    pltpu.sync_copy(zeros_hbm.at[pl.ds(sid * RPT, RPT)],
                    acc_sh.at[pl.ds(sid * RPT, RPT)])
    plsc.subcore_barrier()

    def body(k, carry):
        slot = lax.rem(k, NBUF)
        gather_wait(k, slot)
        scat(k, slot)

        @pl.when(k >= LOOKAHEAD)
        def _():
            kd = k - LOOKAHEAD
            scat_wait(kd, lax.rem(kd, NBUF))

        @pl.when(k < NCHUNK2 - ILOOK)
        def _():
            idx_issue(k + ILOOK)

        @pl.when(k < NCHUNK2 - LOOKAHEAD)
        def _():
            kn = k + LOOKAHEAD
            idx_wait(kn)
            gather(kn, lax.rem(kn, NBUF))
        return carry

    lax.fori_loop(0, NCHUNK2, body, 0)
    for k in range(NCHUNK2 - LOOKAHEAD, NCHUNK2):
        scat_wait(k, k % NBUF)
    plsc.subcore_barrier()
    pltpu.sync_copy(acc_sh.at[pl.ds(sid * RPT, RPT)],
                    out_hbm.at[pl.ds(cid * N_PAD + sid * RPT, RPT)])


BLK = 640
GRID = 16
GRID_S = N_PAD // BLK

_row_spec = pl.BlockSpec((BLK, D), lambda i: (i, 0))
_w_spec = pl.BlockSpec((D, D), lambda i: (0, 0))
_vec_spec = pl.BlockSpec((1, D), lambda i: (0, 0))
_dinv_spec = pl.BlockSpec((BLK, 1), lambda i: (i, 0))
_s0_spec = pl.BlockSpec((BLK, D), lambda i: (i, 0))
_s1_spec = pl.BlockSpec((BLK, D), lambda i: (GRID_S + i, 0))


def _tc_h0_body(x_ref, win_ref, bin_ref, h0_ref):
    h0_ref[...] = jnp.dot(x_ref[...], win_ref[...],
                          preferred_element_type=jnp.float32) + bin_ref[...]


_tc_h0 = pl.pallas_call(
    _tc_h0_body,
    grid=(GRID,),
    in_specs=[_row_spec, _w_spec, _vec_spec],
    out_specs=_row_spec,
    out_shape=jax.ShapeDtypeStruct((N, D), jnp.float32),
)


def _tc0_body(degp_ref, h0_ref, w0_ref, hT0_ref, dinv_ref):
    deg = degp_ref[:, 0:1] + degp_ref[:, 1:2] + 1.0
    dinv = lax.rsqrt(deg)
    hT0_ref[...] = dinv * jnp.dot(h0_ref[...], w0_ref[...],
                                  preferred_element_type=jnp.float32)
    dinv_ref[...] = dinv


_tc0 = pl.pallas_call(
    _tc0_body,
    grid=(GRID,),
    in_specs=[pl.BlockSpec((BLK, NC), lambda i: (i, 0)),
              _row_spec, _w_spec],
    out_specs=[_row_spec, _dinv_spec],
    out_shape=[jax.ShapeDtypeStruct((N, D), jnp.float32),
               jax.ShapeDtypeStruct((N, 1), jnp.float32)],
)


def _post_agg(s0, s1, hT, h, dinv, b, g, be):
    agg = dinv * (s0 + s1 + hT) + b
    mu = jnp.mean(agg, axis=-1, keepdims=True)
    xc = agg - mu
    var = jnp.mean(xc * xc, axis=-1, keepdims=True)
    ln = xc * lax.rsqrt(var + 1e-5) * g + be
    return jnp.maximum(ln, 0.0) + h


def _mid_body(s_ref, s1_ref, hT_ref, h_ref, dinv_ref, b_ref, g_ref, be_ref,
              wn_ref, h1_ref, hT1_ref):
    dinv = dinv_ref[...]
    hn = _post_agg(s_ref[...], s1_ref[...], hT_ref[...], h_ref[...],
                   dinv, b_ref[...], g_ref[...], be_ref[...])
    h1_ref[...] = hn
    hT1_ref[...] = dinv * jnp.dot(hn, wn_ref[...],
                                  preferred_element_type=jnp.float32)


_tc_mid = pl.pallas_call(
    _mid_body,
    grid=(GRID,),
    in_specs=[_s0_spec, _s1_spec, _row_spec, _row_spec, _dinv_spec,
              _vec_spec, _vec_spec, _vec_spec, _w_spec],
    out_specs=[_row_spec, _row_spec],
    out_shape=[jax.ShapeDtypeStruct((N, D), jnp.float32),
               jax.ShapeDtypeStruct((N, D), jnp.float32)],
)


def _fin_body(s_ref, s1_ref, hT_ref, h_ref, dinv_ref, b_ref, g_ref, be_ref,
              h1_ref):
    h1_ref[...] = _post_agg(s_ref[...], s1_ref[...], hT_ref[...], h_ref[...],
                            dinv_ref[...], b_ref[...], g_ref[...], be_ref[...])


_tc_fin = pl.pallas_call(
    _fin_body,
    grid=(GRID,),
    in_specs=[_s0_spec, _s1_spec, _row_spec, _row_spec, _dinv_spec,
              _vec_spec, _vec_spec, _vec_spec],
    out_specs=_row_spec,
    out_shape=jax.ShapeDtypeStruct((N, D), jnp.float32),
)


def kernel(x, edge_index, W_in, b_in, W0, b0, g0, be0,
           W1, b1, g1, be1, W2, b2, g2, be2):
    src = edge_index[0].reshape(NW, NCHUNK2, CH2)
    dst = edge_index[1].reshape(NW, NCHUNK, CH)
    dst2 = edge_index[1].reshape(NW, NCHUNK2, CH2)
    zeros2 = jnp.zeros((N_PAD, D), jnp.float32)
    zeros1 = jnp.zeros((N_PAD,), jnp.float32)

    degp = _deg_kernel(dst, zeros1)
    h = _tc_h0(x, W_in, b_in.reshape(1, D))
    degp2 = jnp.transpose(degp.reshape(NC, N_PAD))
    hT, dinv = _tc0(degp2, h, W0)

    for (b, g, be, Wn) in ((b0, g0, be0, W1), (b1, g1, be1, W2),
                           (b2, g2, be2, None)):
        s = _edge_sum_kernel(src, dst2, hT, zeros2)
        if Wn is None:
            h = _tc_fin(s, s, hT, h, dinv, b.reshape(1, D), g.reshape(1, D),
                        be.reshape(1, D))
        else:
            h, hT = _tc_mid(s, s, hT, h, dinv, b.reshape(1, D),
                            g.reshape(1, D), be.reshape(1, D), Wn)
    return h
